# baseline (speedup 1.0000x reference)
import functools

import jax
import jax.numpy as jnp
from jax import lax
from jax.experimental import pallas as pl
from jax.experimental.pallas import tpu as pltpu
from jax.experimental.pallas import tpu_sc as plsc

N = 50000
PERIODS = 12
HIDDEN = 64
PRED = 12

NSUB = 16
BATCH = 128
LANES = 16

SUP = 7
SLAB_E = 896
NSLAB = 8
GTILE_E = SUP * NSLAB * SLAB_E
GE_PAD = NSUB * GTILE_E

NROWS_PER_SUB = 3200
N_PAD = NSUB * NROWS_PER_SUB
R_SC = N_PAD // 2
T_ROWS = 26624
DUMMY = R_SC
OUT_PER_SUB = R_SC // NSUB

_mesh = plsc.VectorSubcoreMesh(core_axis_name="c", subcore_axis_name="s")
_sc_params = pltpu.CompilerParams(use_tc_tiling_on_sc=False)


def _rsqrt16(v):
    i = lax.bitcast_convert_type(v, jnp.int32)
    seed = jnp.int32(0x5F3759DF) - lax.shift_right_logical(i, 1)
    r = lax.bitcast_convert_type(seed, jnp.float32)
    for _ in range(3):
        r = r * (1.5 - 0.5 * v * r * r)
    return r


def _scale_and_scatter(q, buf, dvb, cidx, wv, t_sh):
    def _scale_grp(m, carry):
        sc = wv[q, pl.ds(m * 16, 16)] * dvb[pl.ds(m * 16, 16)]
        for i in range(16):
            buf[m * 16 + i, :] = buf[m * 16 + i, :] * sc[i]
        return carry

    lax.fori_loop(0, SLAB_E // 16, _scale_grp, 0)
    pltpu.sync_copy(buf, t_sh.at[cidx.at[q]], add=True)


@functools.partial(
    pl.kernel,
    mesh=_mesh,
    compiler_params=_sc_params,
    out_type=(
        jax.ShapeDtypeStruct((2, R_SC, LANES), jnp.float32),
        jax.ShapeDtypeStruct((N_PAD,), jnp.float32),
    ),
    scratch_types=[
        pltpu.VMEM((NSLAB, SLAB_E), jnp.int32),
        pltpu.VMEM((NSLAB, SLAB_E), jnp.int32),
        pltpu.VMEM((NSLAB, SLAB_E), jnp.float32),
        pltpu.VMEM((SLAB_E, LANES), jnp.float32),
        pltpu.VMEM((SLAB_E, LANES), jnp.float32),
        pltpu.VMEM((SLAB_E,), jnp.float32),
        pltpu.VMEM((SLAB_E,), jnp.float32),
        pltpu.VMEM((BATCH, LANES), jnp.float32),
        pltpu.VMEM((NROWS_PER_SUB,), jnp.float32),
        pltpu.VMEM_SHARED((T_ROWS, LANES), jnp.float32),
        pltpu.VMEM_SHARED((N_PAD,), jnp.float32),
        pltpu.SemaphoreType.DMA,
        pltpu.SemaphoreType.DMA,
    ],
)
def _sc_kernel(row_hbm, col_hbm, w_hbm, x_hbm, t_out, dv_out,
               ridx, cidx, wv, rows_a, rows_b, dv_a, dv_b, zblk, dbuf,
               t_sh, deg_sh, sem_a, sem_b):
    c = lax.axis_index("c")
    s = lax.axis_index("s")
    base = c * R_SC

    for i in range(BATCH):
        zblk[i, :] = jnp.zeros((LANES,), jnp.float32)

    def _zero_t(j, carry):
        pltpu.sync_copy(
            zblk, t_sh.at[pl.ds(s * (T_ROWS // NSUB) + j * BATCH, BATCH), :]
        )
        return carry

    lax.fori_loop(0, T_ROWS // NSUB // BATCH, _zero_t, 0)

    def _zero_d(k, carry):
        dbuf[pl.ds(k * 16, 16)] = jnp.zeros((16,), jnp.float32)
        return carry

    lax.fori_loop(0, NROWS_PER_SUB // 16, _zero_d, 0)
    pltpu.sync_copy(dbuf, deg_sh.at[pl.ds(s * NROWS_PER_SUB, NROWS_PER_SUB)])
    plsc.subcore_barrier()

    def _deg_super(u, carry):
        pltpu.sync_copy(col_hbm.at[s, pl.ds(u * NSLAB, NSLAB)], cidx)
        pltpu.sync_copy(w_hbm.at[s, pl.ds(u * NSLAB, NSLAB)], wv)

        def _deg_slab(q, carry2):
            pltpu.sync_copy(wv.at[q], deg_sh.at[cidx.at[q]], add=True)
            return carry2

        lax.fori_loop(0, NSLAB, _deg_slab, 0)
        return carry

    lax.fori_loop(0, SUP, _deg_super, 0)
    plsc.subcore_barrier()

    pltpu.sync_copy(deg_sh.at[pl.ds(s * NROWS_PER_SUB, NROWS_PER_SUB)], dbuf)

    def _rs(k, carry):
        v = dbuf[pl.ds(k * 16, 16)] + 1.0
        dbuf[pl.ds(k * 16, 16)] = _rsqrt16(v)
        return carry

    lax.fori_loop(0, NROWS_PER_SUB // 16, _rs, 0)
    pltpu.sync_copy(dbuf, deg_sh.at[pl.ds(s * NROWS_PER_SUB, NROWS_PER_SUB)])
    pltpu.sync_copy(dbuf, dv_out.at[pl.ds(s * NROWS_PER_SUB, NROWS_PER_SUB)])
    plsc.subcore_barrier()

    def _super(u, carry):
        pltpu.sync_copy(row_hbm.at[s, pl.ds(u * NSLAB, NSLAB)], ridx)
        pltpu.sync_copy(col_hbm.at[s, pl.ds(u * NSLAB, NSLAB)], cidx)
        pltpu.sync_copy(w_hbm.at[s, pl.ds(u * NSLAB, NSLAB)], wv)

        def _mask(m, carry2):
            q = m // (SLAB_E // 16)
            o = (m % (SLAB_E // 16)) * 16
            v = cidx[q, pl.ds(o, 16)]
            local = v - base
            ok = (local >= 0) & (local < R_SC)
            cidx[q, pl.ds(o, 16)] = jnp.where(ok, local, DUMMY)
            return carry2

        lax.fori_loop(0, NSLAB * (SLAB_E // 16), _mask, 0)

        def _fire(q, rows, dvv, sem):
            pltpu.async_copy(x_hbm.at[ridx.at[q]], rows, sem)
            pltpu.async_copy(dv_out.at[ridx.at[q]], dvv, sem)

        def _wait(q, rows, dvv, sem):
            pltpu.make_async_copy(x_hbm.at[ridx.at[q]], rows, sem).wait()
            pltpu.make_async_copy(dv_out.at[ridx.at[q]], dvv, sem).wait()

        _fire(0, rows_a, dv_a, sem_a)

        def _pair(jj, carry2):
            q0 = jj * 2
            _fire(q0 + 1, rows_b, dv_b, sem_b)
            _wait(q0, rows_a, dv_a, sem_a)
            _scale_and_scatter(q0, rows_a, dv_a, cidx, wv, t_sh)
            _fire(q0 + 2, rows_a, dv_a, sem_a)
            _wait(q0 + 1, rows_b, dv_b, sem_b)
            _scale_and_scatter(q0 + 1, rows_b, dv_b, cidx, wv, t_sh)
            return carry2

        lax.fori_loop(0, NSLAB // 2 - 1, _pair, 0)

        qt = NSLAB - 2
        _fire(qt + 1, rows_b, dv_b, sem_b)
        _wait(qt, rows_a, dv_a, sem_a)
        _scale_and_scatter(qt, rows_a, dv_a, cidx, wv, t_sh)
        _wait(qt + 1, rows_b, dv_b, sem_b)
        _scale_and_scatter(qt + 1, rows_b, dv_b, cidx, wv, t_sh)
        return carry

    lax.fori_loop(0, SUP, _super, 0)
    plsc.subcore_barrier()

    pltpu.sync_copy(
        t_sh.at[pl.ds(s * OUT_PER_SUB, OUT_PER_SUB), :],
        t_out.at[c, pl.ds(s * OUT_PER_SUB, OUT_PER_SUB), :],
    )


def _dense_body(t, x, dv, az, ah, bz, bh, ow, ob, out_ref, h0_ref):
    dvv = dv[...]
    s12 = dvv * t[:, :PERIODS] + (dvv * dvv) * x[...]
    z = jax.nn.sigmoid(
        jnp.dot(s12, az[...], preferred_element_type=jnp.float32,
                precision=lax.Precision.HIGHEST) + bz[...]
    )
    ht = jnp.tanh(
        jnp.dot(s12, ah[...], preferred_element_type=jnp.float32,
                precision=lax.Precision.HIGHEST) + bh[...]
    )
    h0 = (1.0 - z) * ht
    h0_ref[...] = h0
    out_ref[...] = (
        jnp.dot(jax.nn.relu(h0), ow[...], preferred_element_type=jnp.float32,
                precision=lax.Precision.HIGHEST)
        + ob[...]
    )


def kernel(x, edge_index, edge_weight, W_z, b_z, lz_W, lz_b, W_r, b_r, lr_W,
           lr_b, W_h, b_h, lh_W, lh_b, out_W, out_b):
    row = edge_index[0]
    col = edge_index[1]
    e = row.shape[0]
    pad_e = GE_PAD - e
    rowf = jnp.concatenate([row, jnp.zeros((pad_e,), jnp.int32)])
    colf = jnp.concatenate([col, jnp.zeros((pad_e,), jnp.int32)])
    wf = jnp.concatenate([edge_weight, jnp.zeros((pad_e,), jnp.float32)])
    x_pad = jnp.zeros((N_PAD, LANES), jnp.float32).at[:N, :PERIODS].set(x)

    t_p, dv_flat = _sc_kernel(
        rowf.reshape(NSUB, SUP * NSLAB, SLAB_E),
        colf.reshape(NSUB, SUP * NSLAB, SLAB_E),
        wf.reshape(NSUB, SUP * NSLAB, SLAB_E), x_pad)
    t_full = t_p.reshape(N_PAD, LANES)
    dv2 = dv_flat.reshape(N_PAD, 1)

    az = W_z @ lz_W[:HIDDEN]
    ah = W_h @ lh_W[:HIDDEN]
    bz2 = (b_z @ lz_W[:HIDDEN] + lz_b).reshape(1, HIDDEN)
    bh2 = (b_h @ lh_W[:HIDDEN] + lh_b).reshape(1, HIDDEN)
    ob = out_b.reshape(1, PRED)

    blk = NROWS_PER_SUB
    grid4 = (N + blk - 1) // blk
    out, h0 = pl.pallas_call(
        _dense_body,
        grid=(grid4,),
        in_specs=[
            pl.BlockSpec((blk, LANES), lambda i: (i, 0)),
            pl.BlockSpec((blk, PERIODS), lambda i: (i, 0)),
            pl.BlockSpec((blk, 1), lambda i: (i, 0)),
            pl.BlockSpec((PERIODS, HIDDEN), lambda i: (0, 0)),
            pl.BlockSpec((PERIODS, HIDDEN), lambda i: (0, 0)),
            pl.BlockSpec((1, HIDDEN), lambda i: (0, 0)),
            pl.BlockSpec((1, HIDDEN), lambda i: (0, 0)),
            pl.BlockSpec((HIDDEN, PRED), lambda i: (0, 0)),
            pl.BlockSpec((1, PRED), lambda i: (0, 0)),
        ],
        out_specs=[
            pl.BlockSpec((blk, PRED), lambda i: (i, 0)),
            pl.BlockSpec((blk, HIDDEN), lambda i: (i, 0)),
        ],
        out_shape=[
            jax.ShapeDtypeStruct((N, PRED), jnp.float32),
            jax.ShapeDtypeStruct((N, HIDDEN), jnp.float32),
        ],
    )(t_full, x_pad[:, :PERIODS], dv2, az, ah, bz2, bh2, out_W, ob)
    return (out, h0)


# speedup vs baseline: 1.0110x; 1.0110x over previous
import functools

import jax
import jax.numpy as jnp
from jax import lax
from jax.experimental import pallas as pl
from jax.experimental.pallas import tpu as pltpu
from jax.experimental.pallas import tpu_sc as plsc

N = 50000
PERIODS = 12
HIDDEN = 64
PRED = 12

NSUB = 16
BATCH = 128
LANES = 16

SUP = 25
NSLAB = 5
SLAB_E = 400
GTILE_E = SUP * NSLAB * SLAB_E

NROWS_PER_SUB = 3200
N_PAD = NSUB * NROWS_PER_SUB
R_SC = N_PAD // 2
T_ROWS = 26624
DUMMY = R_SC
OUT_PER_SUB = R_SC // NSUB

_mesh = plsc.VectorSubcoreMesh(core_axis_name="c", subcore_axis_name="s")
_sc_params = pltpu.CompilerParams(use_tc_tiling_on_sc=False)


def _rsqrt16(v):
    i = lax.bitcast_convert_type(v, jnp.int32)
    seed = jnp.int32(0x5F3759DF) - lax.shift_right_logical(i, 1)
    r = lax.bitcast_convert_type(seed, jnp.float32)
    for _ in range(3):
        r = r * (1.5 - 0.5 * v * r * r)
    return r


def _scale_and_scatter(q, buf, dvb, cidx, wv, t_sh):
    def _scale_grp(m, carry):
        sc = wv[q, pl.ds(m * 16, 16)] * dvb[pl.ds(m * 16, 16)]
        for i in range(16):
            buf[m * 16 + i, :] = buf[m * 16 + i, :] * sc[i]
        return carry

    lax.fori_loop(0, SLAB_E // 16, _scale_grp, 0)
    pltpu.sync_copy(buf, t_sh.at[cidx.at[q]], add=True)


@functools.partial(
    pl.kernel,
    mesh=_mesh,
    compiler_params=_sc_params,
    out_type=(
        jax.ShapeDtypeStruct((2, R_SC, LANES), jnp.float32),
        jax.ShapeDtypeStruct((N_PAD,), jnp.float32),
    ),
    scratch_types=[
        pltpu.VMEM((NSLAB, SLAB_E), jnp.int32),
        pltpu.VMEM((NSLAB, SLAB_E), jnp.int32),
        pltpu.VMEM((NSLAB, SLAB_E), jnp.float32),
        pltpu.VMEM((SLAB_E, LANES), jnp.float32),
        pltpu.VMEM((SLAB_E, LANES), jnp.float32),
        pltpu.VMEM((SLAB_E,), jnp.float32),
        pltpu.VMEM((SLAB_E,), jnp.float32),
        pltpu.VMEM((BATCH, LANES), jnp.float32),
        pltpu.VMEM((NROWS_PER_SUB,), jnp.float32),
        pltpu.VMEM_SHARED((T_ROWS, LANES), jnp.float32),
        pltpu.VMEM_SHARED((N_PAD,), jnp.float32),
        pltpu.SemaphoreType.DMA,
        pltpu.SemaphoreType.DMA,
    ],
)
def _sc_kernel(row_hbm, col_hbm, w_hbm, x_hbm, t_out, dv_out,
               ridx, cidx, wv, rows_a, rows_b, dv_a, dv_b, zblk, dbuf,
               t_sh, deg_sh, sem_a, sem_b):
    c = lax.axis_index("c")
    s = lax.axis_index("s")
    base = c * R_SC

    for i in range(BATCH):
        zblk[i, :] = jnp.zeros((LANES,), jnp.float32)

    def _zero_t(j, carry):
        pltpu.sync_copy(
            zblk, t_sh.at[pl.ds(s * (T_ROWS // NSUB) + j * BATCH, BATCH), :]
        )
        return carry

    lax.fori_loop(0, T_ROWS // NSUB // BATCH, _zero_t, 0)

    def _zero_d(k, carry):
        dbuf[pl.ds(k * 16, 16)] = jnp.zeros((16,), jnp.float32)
        return carry

    lax.fori_loop(0, NROWS_PER_SUB // 16, _zero_d, 0)
    pltpu.sync_copy(dbuf, deg_sh.at[pl.ds(s * NROWS_PER_SUB, NROWS_PER_SUB)])
    plsc.subcore_barrier()

    def _deg_super(u, carry):
        pltpu.sync_copy(col_hbm.at[s, pl.ds(u * NSLAB, NSLAB)], cidx)
        pltpu.sync_copy(w_hbm.at[s, pl.ds(u * NSLAB, NSLAB)], wv)

        def _deg_slab(q, carry2):
            pltpu.sync_copy(wv.at[q], deg_sh.at[cidx.at[q]], add=True)
            return carry2

        lax.fori_loop(0, NSLAB, _deg_slab, 0)
        return carry

    lax.fori_loop(0, SUP, _deg_super, 0)
    plsc.subcore_barrier()

    pltpu.sync_copy(deg_sh.at[pl.ds(s * NROWS_PER_SUB, NROWS_PER_SUB)], dbuf)

    def _rs(k, carry):
        v = dbuf[pl.ds(k * 16, 16)] + 1.0
        dbuf[pl.ds(k * 16, 16)] = _rsqrt16(v)
        return carry

    lax.fori_loop(0, NROWS_PER_SUB // 16, _rs, 0)
    pltpu.sync_copy(dbuf, deg_sh.at[pl.ds(s * NROWS_PER_SUB, NROWS_PER_SUB)])
    pltpu.sync_copy(dbuf, dv_out.at[pl.ds(s * NROWS_PER_SUB, NROWS_PER_SUB)])
    plsc.subcore_barrier()

    def _super(u, carry):
        pltpu.sync_copy(row_hbm.at[s, pl.ds(u * NSLAB, NSLAB)], ridx)
        pltpu.sync_copy(col_hbm.at[s, pl.ds(u * NSLAB, NSLAB)], cidx)
        pltpu.sync_copy(w_hbm.at[s, pl.ds(u * NSLAB, NSLAB)], wv)

        def _mask(m, carry2):
            q = m // (SLAB_E // 16)
            o = (m % (SLAB_E // 16)) * 16
            v = cidx[q, pl.ds(o, 16)]
            local = v - base
            ok = (local >= 0) & (local < R_SC)
            cidx[q, pl.ds(o, 16)] = jnp.where(ok, local, DUMMY)
            return carry2

        lax.fori_loop(0, NSLAB * (SLAB_E // 16), _mask, 0)

        def _fire(q, rows, dvv, sem):
            pltpu.async_copy(x_hbm.at[ridx.at[q]], rows, sem)
            pltpu.async_copy(dv_out.at[ridx.at[q]], dvv, sem)

        def _wait(q, rows, dvv, sem):
            pltpu.make_async_copy(x_hbm.at[ridx.at[q]], rows, sem).wait()
            pltpu.make_async_copy(dv_out.at[ridx.at[q]], dvv, sem).wait()

        _fire(0, rows_a, dv_a, sem_a)

        def _pair(jj, carry2):
            q0 = jj * 2
            _fire(q0 + 1, rows_b, dv_b, sem_b)
            _wait(q0, rows_a, dv_a, sem_a)
            _scale_and_scatter(q0, rows_a, dv_a, cidx, wv, t_sh)
            _fire(q0 + 2, rows_a, dv_a, sem_a)
            _wait(q0 + 1, rows_b, dv_b, sem_b)
            _scale_and_scatter(q0 + 1, rows_b, dv_b, cidx, wv, t_sh)
            return carry2

        lax.fori_loop(0, (NSLAB - 1) // 2, _pair, 0)

        qt = NSLAB - 1
        _wait(qt, rows_a, dv_a, sem_a)
        _scale_and_scatter(qt, rows_a, dv_a, cidx, wv, t_sh)
        return carry

    lax.fori_loop(0, SUP, _super, 0)
    plsc.subcore_barrier()

    pltpu.sync_copy(
        t_sh.at[pl.ds(s * OUT_PER_SUB, OUT_PER_SUB), :],
        t_out.at[c, pl.ds(s * OUT_PER_SUB, OUT_PER_SUB), :],
    )


def _dense_body(t, x, dv, wz, lz, bz, lzb, wh, lh, bh, lhb, ow, ob,
                out_ref, h0_ref):
    hp = lax.Precision.HIGHEST
    az = jnp.dot(wz[...], lz[:HIDDEN, :], preferred_element_type=jnp.float32,
                 precision=hp)
    ah = jnp.dot(wh[...], lh[:HIDDEN, :], preferred_element_type=jnp.float32,
                 precision=hp)
    bz2 = jnp.dot(bz[...], lz[:HIDDEN, :], preferred_element_type=jnp.float32,
                  precision=hp) + lzb[...]
    bh2 = jnp.dot(bh[...], lh[:HIDDEN, :], preferred_element_type=jnp.float32,
                  precision=hp) + lhb[...]
    dvv = dv[...]
    s12 = dvv * t[0, :, :PERIODS] + (dvv * dvv) * x[...]
    z = jax.nn.sigmoid(
        jnp.dot(s12, az, preferred_element_type=jnp.float32, precision=hp)
        + bz2
    )
    ht = jnp.tanh(
        jnp.dot(s12, ah, preferred_element_type=jnp.float32, precision=hp)
        + bh2
    )
    h0 = (1.0 - z) * ht
    h0_ref[...] = h0
    out_ref[...] = (
        jnp.dot(jax.nn.relu(h0), ow[...], preferred_element_type=jnp.float32,
                precision=hp)
        + ob[...]
    )


def kernel(x, edge_index, edge_weight, W_z, b_z, lz_W, lz_b, W_r, b_r, lr_W,
           lr_b, W_h, b_h, lh_W, lh_b, out_W, out_b):
    rowp = edge_index[0].reshape(NSUB, SUP * NSLAB, SLAB_E)
    colp = edge_index[1].reshape(NSUB, SUP * NSLAB, SLAB_E)
    wp = edge_weight.reshape(NSUB, SUP * NSLAB, SLAB_E)
    x_pad = jnp.zeros((N_PAD, LANES), jnp.float32).at[:N, :PERIODS].set(x)

    t_p, dv_flat = _sc_kernel(rowp, colp, wp, x_pad)
    dv2 = dv_flat.reshape(N_PAD, 1)

    blk = NROWS_PER_SUB
    nblk_half = R_SC // blk
    grid4 = (N + blk - 1) // blk
    out, h0 = pl.pallas_call(
        _dense_body,
        grid=(grid4,),
        in_specs=[
            pl.BlockSpec((1, blk, LANES),
                         lambda i: (i // nblk_half, i % nblk_half, 0)),
            pl.BlockSpec((blk, PERIODS), lambda i: (i, 0)),
            pl.BlockSpec((blk, 1), lambda i: (i, 0)),
            pl.BlockSpec((PERIODS, HIDDEN), lambda i: (0, 0)),
            pl.BlockSpec((2 * HIDDEN, HIDDEN), lambda i: (0, 0)),
            pl.BlockSpec((1, HIDDEN), lambda i: (0, 0)),
            pl.BlockSpec((1, HIDDEN), lambda i: (0, 0)),
            pl.BlockSpec((PERIODS, HIDDEN), lambda i: (0, 0)),
            pl.BlockSpec((2 * HIDDEN, HIDDEN), lambda i: (0, 0)),
            pl.BlockSpec((1, HIDDEN), lambda i: (0, 0)),
            pl.BlockSpec((1, HIDDEN), lambda i: (0, 0)),
            pl.BlockSpec((HIDDEN, PRED), lambda i: (0, 0)),
            pl.BlockSpec((1, PRED), lambda i: (0, 0)),
        ],
        out_specs=[
            pl.BlockSpec((blk, PRED), lambda i: (i, 0)),
            pl.BlockSpec((blk, HIDDEN), lambda i: (i, 0)),
        ],
        out_shape=[
            jax.ShapeDtypeStruct((N, PRED), jnp.float32),
            jax.ShapeDtypeStruct((N, HIDDEN), jnp.float32),
        ],
    )(t_p, x, dv2, W_z, lz_W, b_z.reshape(1, HIDDEN), lz_b.reshape(1, HIDDEN),
      W_h, lh_W, b_h.reshape(1, HIDDEN), lh_b.reshape(1, HIDDEN), out_W,
      out_b.reshape(1, PRED))
    return (out, h0)
